# baseline (speedup 1.0000x reference)
import dataclasses
import functools

import jax
import jax.numpy as jnp
from jax import lax
from jax.experimental import pallas as pl
from jax.experimental.pallas import tpu as pltpu
from jax.experimental.pallas import tpu_sc as plsc

NUM_LEVELS = 32
LEVEL_DIM = 2
LOG2_HASHMAP = 19
T = 2 ** LOG2_HASHMAP
P1 = 2654435761
P2 = 805459861
N_POINTS = 262144

NUM_CORES = 2
NUM_SUBCORES = 16
NUM_WORKERS = NUM_CORES * NUM_SUBCORES
PTS_PER_WORKER = N_POINTS // NUM_WORKERS
CHUNK = 512
NCHUNKS = PTS_PER_WORKER // CHUNK
ROWS = 8 * CHUNK
LANES = 16
NBLK = NUM_LEVELS * T // 8
NBLK8 = NUM_LEVELS * T // 4

MLP_BLOCK = 2048


def _mesh_and_params():
    mesh = plsc.VectorSubcoreMesh(
        core_axis_name="c", subcore_axis_name="s",
        num_cores=NUM_CORES, num_subcores=NUM_SUBCORES)
    cp = pltpu.CompilerParams()
    if "needs_layout_passes" in pltpu.CompilerParams.__dataclass_fields__:
        cp = dataclasses.replace(cp, needs_layout_passes=False)
    cp = dataclasses.replace(cp, use_tc_tiling_on_sc=False)
    return mesh, cp


REL_ROWS = 1024


def _sc_relayout(tabn):
    mesh, cp = _mesh_and_params()
    per_tile = NBLK // NUM_WORKERS
    nchunks = per_tile // REL_ROWS

    @functools.partial(
        pl.kernel,
        out_type=jax.ShapeDtypeStruct((NBLK8, 8), jnp.float32),
        mesh=mesh,
        scratch_types=[
            pltpu.VMEM((REL_ROWS, 16), jnp.float32),
            pltpu.VMEM((REL_ROWS * 2, 8), jnp.float32),
            pltpu.SemaphoreType.DMA,
        ],
        compiler_params=cp,
    )
    def rel_kernel(tn_hbm, ti_hbm, nbuf, ibuf, sem):
        cid = lax.axis_index("c")
        sid = lax.axis_index("s")
        wid = sid * NUM_CORES + cid
        base_w = wid * per_tile
        lanes = lax.iota(jnp.int32, LANES)
        row_off = lanes >> 2
        lane_off = (lanes * 2) & 7

        @pl.loop(0, nchunks)
        def _chunk(ci):
            blk0 = base_w + ci * REL_ROWS
            pltpu.async_copy(tn_hbm.at[pl.ds(blk0, REL_ROWS)], nbuf,
                             sem).wait()

            @pl.loop(0, REL_ROWS // 16)
            def _grp(g):
                for f in range(2):
                    for k in range(8):
                        v = nbuf[g * 16 + f * 8 + k, :]
                        plsc.store_scatter(
                            ibuf,
                            [g * 32 + 4 * k + row_off, lane_off + f], v)

            pltpu.async_copy(ibuf, ti_hbm.at[pl.ds(blk0 * 2, REL_ROWS * 2)],
                             sem).wait()

    return rel_kernel(tabn)


def _sc_encode(xt, tabflat):
    mesh, cp = _mesh_and_params()

    @functools.partial(
        pl.kernel,
        out_type=jax.ShapeDtypeStruct((NUM_LEVELS * LEVEL_DIM, N_POINTS),
                                      jnp.float32),
        mesh=mesh,
        scratch_types=[
            pltpu.VMEM((3, CHUNK), jnp.float32),
            pltpu.VMEM((ROWS,), jnp.int32),
            pltpu.VMEM((ROWS,), jnp.int32),
            pltpu.VMEM((ROWS,), jnp.int32),
            pltpu.VMEM((ROWS,), jnp.int32),
            pltpu.VMEM((ROWS,), jnp.float32),
            pltpu.VMEM((ROWS,), jnp.float32),
            pltpu.VMEM((ROWS, 8), jnp.float32),
            pltpu.VMEM((ROWS, 8), jnp.float32),
            pltpu.VMEM((LEVEL_DIM, CHUNK), jnp.float32),
            pltpu.VMEM((LEVEL_DIM, CHUNK), jnp.float32),
            pltpu.SemaphoreType.DMA,
            pltpu.SemaphoreType.DMA,
            pltpu.SemaphoreType.DMA,
            pltpu.SemaphoreType.DMA,
        ],
        compiler_params=cp,
    )
    def enc_kernel(xt_hbm, tab_hbm, enc_hbm, xyz, idxb0, idxb1, laneb0,
                   laneb1, wb0, wb1, gb0, gb1, encst0, encst1, gsem0, gsem1,
                   esem0, esem1):
        cid = lax.axis_index("c")
        sid = lax.axis_index("s")
        wid = sid * NUM_CORES + cid
        base_w = wid * PTS_PER_WORKER
        lanes = lax.iota(jnp.int32, LANES)
        idxbs = (idxb0, idxb1)
        lanebs = (laneb0, laneb1)
        wbs = (wb0, wb1)
        gbs = (gb0, gb1)
        encsts = (encst0, encst1)
        gsems = (gsem0, gsem1)
        esems = (esem0, esem1)

        def hash_fire(l, par):
            idxb, laneb, wb, gb = idxbs[par], lanebs[par], wbs[par], gbs[par]
            lt8 = l * (T // 4)
            scale = lax.bitcast_convert_type(
                jnp.full((LANES,), (l + 127) << 23, jnp.int32), jnp.float32)
            @plsc.parallel_loop(0, CHUNK, step=LANES)
            def _grp(p):
                xv = xyz[0, pl.ds(p, LANES)]
                yv = xyz[1, pl.ds(p, LANES)]
                zv = xyz[2, pl.ds(p, LANES)]
                px = xv * scale
                py = yv * scale
                pz = zv * scale
                ix = px.astype(jnp.uint32)
                iy = py.astype(jnp.uint32)
                iz = pz.astype(jnp.uint32)
                fx = px - ix.astype(jnp.float32)
                fy = py - iy.astype(jnp.float32)
                fz = pz - iz.astype(jnp.float32)
                one = jnp.float32(1.0)
                wxs = (one - fx, fx)
                wys = (one - fy, fy)
                wzs = (one - fz, fz)
                hx = (ix, ix + jnp.uint32(1))
                hy0 = iy * jnp.uint32(P1)
                hys = (hy0, hy0 + jnp.uint32(P1))
                hz0 = iz * jnp.uint32(P2)
                hzs = (hz0, hz0 + jnp.uint32(P2))
                for corner in range(8):
                    bx = corner & 1
                    by = (corner >> 1) & 1
                    bz = (corner >> 2) & 1
                    h = hx[bx] ^ hys[by] ^ hzs[bz]
                    hm = lax.bitcast_convert_type(
                        h & jnp.uint32(T - 1), jnp.int32)
                    w = (wxs[bx] * wys[by]) * wzs[bz]
                    row = corner * CHUNK + p
                    idxb[pl.ds(row, LANES)] = (hm >> 2) + lt8
                    laneb[pl.ds(row, LANES)] = (hm & 3) * 2
                    wb[pl.ds(row, LANES)] = w

            pltpu.async_copy(tab_hbm.at[idxb], gb, gsems[par])

        def acc_write(l, par, base):
            idxb, laneb, wb, gb = idxbs[par], lanebs[par], wbs[par], gbs[par]
            encst = encsts[par]
            enc_dst = enc_hbm.at[pl.ds(l * LEVEL_DIM, LEVEL_DIM),
                                 pl.ds(base, CHUNK)]
            @pl.when(l >= 2)
            def _():
                pltpu.make_async_copy(encst, enc_dst, esems[par]).wait()

            pltpu.make_async_copy(tab_hbm.at[idxb], gb, gsems[par]).wait()

            @plsc.parallel_loop(0, CHUNK, step=LANES)
            def _acc(p):
                acc0 = jnp.zeros((LANES,), jnp.float32)
                acc1 = jnp.zeros((LANES,), jnp.float32)
                for corner in range(8):
                    row = corner * CHUNK + p
                    rows = row + lanes
                    w = wb[pl.ds(row, LANES)]
                    lv = laneb[pl.ds(row, LANES)]
                    g0 = plsc.load_gather(gb, [rows, lv])
                    g1 = plsc.load_gather(gb, [rows, lv + 1])
                    acc0 = acc0 + w * g0
                    acc1 = acc1 + w * g1
                encst[0, pl.ds(p, LANES)] = acc0
                encst[1, pl.ds(p, LANES)] = acc1

            pltpu.async_copy(encst, enc_dst, esems[par])

        @pl.loop(0, NCHUNKS)
        def _chunk(ci):
            base = base_w + ci * CHUNK
            pltpu.sync_copy(xt_hbm.at[pl.ds(0, 3), pl.ds(base, CHUNK)], xyz)
            hash_fire(jnp.int32(0), 0)

            @pl.loop(0, NUM_LEVELS // 2)
            def _lp(lp):
                l0 = lp * 2
                hash_fire(l0 + 1, 1)
                acc_write(l0, 0, base)

                @pl.when(lp <= NUM_LEVELS // 2 - 2)
                def _():
                    hash_fire(l0 + 2, 0)

                acc_write(l0 + 1, 1, base)

            for par in range(2):
                l_last = NUM_LEVELS - 2 + par
                enc_dst = enc_hbm.at[pl.ds(l_last * LEVEL_DIM, LEVEL_DIM),
                                     pl.ds(base, CHUNK)]
                pltpu.make_async_copy(encsts[par], enc_dst,
                                      esems[par]).wait()

    return enc_kernel(xt, tabflat)


def _mlp_body(enc_ref, w0_ref, b0_ref, w1_ref, b1_ref, w2_ref, b2_ref,
              out_ref):
    e = enc_ref[...]
    dn = (((0,), (0,)), ((), ()))
    h0 = lax.dot_general(w0_ref[...], e, dn,
                         preferred_element_type=jnp.float32) + b0_ref[...]
    h0 = jnp.maximum(h0, 0.0)
    h1 = lax.dot_general(w1_ref[...], h0, dn,
                         preferred_element_type=jnp.float32) + b1_ref[...]
    h1 = jnp.maximum(h1, 0.0)
    out_ref[...] = lax.dot_general(w2_ref[...], h1, dn,
                                   preferred_element_type=jnp.float32) \
        + b2_ref[...]


def _tc_mlp(enc, W0, b0, W1, b1, W2, b2):
    n = enc.shape[1]
    grid = (n // MLP_BLOCK,)
    return pl.pallas_call(
        _mlp_body,
        grid=grid,
        in_specs=[
            pl.BlockSpec((enc.shape[0], MLP_BLOCK), lambda i: (0, i)),
            pl.BlockSpec(W0.shape, lambda i: (0, 0)),
            pl.BlockSpec((128, 1), lambda i: (0, 0)),
            pl.BlockSpec(W1.shape, lambda i: (0, 0)),
            pl.BlockSpec((128, 1), lambda i: (0, 0)),
            pl.BlockSpec(W2.shape, lambda i: (0, 0)),
            pl.BlockSpec((1, 1), lambda i: (0, 0)),
        ],
        out_specs=pl.BlockSpec((1, MLP_BLOCK), lambda i: (0, i)),
        out_shape=jax.ShapeDtypeStruct((1, n), jnp.float32),
    )(enc, W0, b0.reshape(128, 1), W1, b1.reshape(128, 1), W2,
      b2.reshape(1, 1))


@jax.jit
def kernel(x, table, W0, b0, W1, b1, W2, b2):
    xt = x.T
    tabn = (table.reshape(NUM_LEVELS, T // 128, 128, LEVEL_DIM)
            .transpose(0, 1, 3, 2).reshape(NBLK, 16))
    tabflat = _sc_relayout(tabn)
    enc = _sc_encode(xt, tabflat)
    out = _tc_mlp(enc, W0, b0, W1, b1, W2, b2)
    return out.reshape(-1, 1)


# speedup vs baseline: 1.0276x; 1.0276x over previous
import dataclasses
import functools

import jax
import jax.numpy as jnp
from jax import lax
from jax.experimental import pallas as pl
from jax.experimental.pallas import tpu as pltpu
from jax.experimental.pallas import tpu_sc as plsc

NUM_LEVELS = 32
LEVEL_DIM = 2
LOG2_HASHMAP = 19
T = 2 ** LOG2_HASHMAP
P1 = 2654435761
P2 = 805459861
N_POINTS = 262144

NUM_CORES = 2
NUM_SUBCORES = 16
NUM_WORKERS = NUM_CORES * NUM_SUBCORES
PTS_PER_WORKER = N_POINTS // NUM_WORKERS
CHUNK = 512
NCHUNKS = PTS_PER_WORKER // CHUNK
ROWS = 8 * CHUNK
LANES = 16
NBLK = NUM_LEVELS * T // 8
NBLK8 = NUM_LEVELS * T // 4

MLP_BLOCK = 2048


def _mesh_and_params():
    mesh = plsc.VectorSubcoreMesh(
        core_axis_name="c", subcore_axis_name="s",
        num_cores=NUM_CORES, num_subcores=NUM_SUBCORES)
    cp = pltpu.CompilerParams()
    if "needs_layout_passes" in pltpu.CompilerParams.__dataclass_fields__:
        cp = dataclasses.replace(cp, needs_layout_passes=False)
    cp = dataclasses.replace(cp, use_tc_tiling_on_sc=False)
    return mesh, cp


REL_ROWS = 1024


def _sc_relayout(tabn):
    mesh, cp = _mesh_and_params()
    per_tile = NBLK // NUM_WORKERS
    nchunks = per_tile // REL_ROWS

    @functools.partial(
        pl.kernel,
        out_type=jax.ShapeDtypeStruct((NBLK8, 8), jnp.float32),
        mesh=mesh,
        scratch_types=[
            pltpu.VMEM((REL_ROWS, 16), jnp.float32),
            pltpu.VMEM((REL_ROWS, 16), jnp.float32),
            pltpu.VMEM((REL_ROWS * 2, 8), jnp.float32),
            pltpu.VMEM((REL_ROWS * 2, 8), jnp.float32),
            pltpu.SemaphoreType.DMA,
            pltpu.SemaphoreType.DMA,
            pltpu.SemaphoreType.DMA,
            pltpu.SemaphoreType.DMA,
        ],
        compiler_params=cp,
    )
    def rel_kernel(tn_hbm, ti_hbm, nbuf0, nbuf1, ibuf0, ibuf1, isem0, isem1,
                   osem0, osem1):
        cid = lax.axis_index("c")
        sid = lax.axis_index("s")
        wid = sid * NUM_CORES + cid
        base_w = wid * per_tile
        lanes = lax.iota(jnp.int32, LANES)
        row_off = lanes >> 2
        lane_off = (lanes * 2) & 7
        nbufs = (nbuf0, nbuf1)
        ibufs = (ibuf0, ibuf1)
        isems = (isem0, isem1)
        osems = (osem0, osem1)

        def src(c):
            return tn_hbm.at[pl.ds(base_w + c * REL_ROWS, REL_ROWS)]

        def dst(c):
            return ti_hbm.at[pl.ds((base_w + c * REL_ROWS) * 2,
                                   REL_ROWS * 2)]

        pltpu.async_copy(src(0), nbuf0, isem0)

        @pl.loop(0, nchunks // 2)
        def _chunk(cc):
            for par in range(2):
                c = cc * 2 + par
                nbuf, ibuf = nbufs[par], ibufs[par]

                @pl.when(c + 1 <= nchunks - 1)
                def _():
                    pltpu.async_copy(src(c + 1), nbufs[1 - par],
                                     isems[1 - par])

                pltpu.make_async_copy(src(c), nbuf, isems[par]).wait()

                @pl.when(c >= 2)
                def _():
                    pltpu.make_async_copy(ibuf, dst(c), osems[par]).wait()

                @pl.loop(0, REL_ROWS // 16)
                def _grp(g):
                    for f in range(2):
                        for k in range(8):
                            v = nbuf[g * 16 + f * 8 + k, :]
                            plsc.store_scatter(
                                ibuf,
                                [g * 32 + 4 * k + row_off, lane_off + f], v)

                pltpu.async_copy(ibuf, dst(c), osems[par])

        for par in range(2):
            pltpu.make_async_copy(ibufs[par], dst(nchunks - 2 + par),
                                  osems[par]).wait()

    return rel_kernel(tabn)


def _sc_encode(xt, tabflat):
    mesh, cp = _mesh_and_params()

    @functools.partial(
        pl.kernel,
        out_type=jax.ShapeDtypeStruct((NUM_LEVELS * LEVEL_DIM, N_POINTS),
                                      jnp.float32),
        mesh=mesh,
        scratch_types=[
            pltpu.VMEM((3, CHUNK), jnp.float32),
            pltpu.VMEM((ROWS,), jnp.int32),
            pltpu.VMEM((ROWS,), jnp.int32),
            pltpu.VMEM((ROWS,), jnp.int32),
            pltpu.VMEM((ROWS,), jnp.int32),
            pltpu.VMEM((ROWS,), jnp.float32),
            pltpu.VMEM((ROWS,), jnp.float32),
            pltpu.VMEM((ROWS, 8), jnp.float32),
            pltpu.VMEM((ROWS, 8), jnp.float32),
            pltpu.VMEM((LEVEL_DIM, CHUNK), jnp.float32),
            pltpu.VMEM((LEVEL_DIM, CHUNK), jnp.float32),
            pltpu.SemaphoreType.DMA,
            pltpu.SemaphoreType.DMA,
            pltpu.SemaphoreType.DMA,
            pltpu.SemaphoreType.DMA,
        ],
        compiler_params=cp,
    )
    def enc_kernel(xt_hbm, tab_hbm, enc_hbm, xyz, idxb0, idxb1, laneb0,
                   laneb1, wb0, wb1, gb0, gb1, encst0, encst1, gsem0, gsem1,
                   esem0, esem1):
        cid = lax.axis_index("c")
        sid = lax.axis_index("s")
        wid = sid * NUM_CORES + cid
        base_w = wid * PTS_PER_WORKER
        lanes = lax.iota(jnp.int32, LANES)
        idxbs = (idxb0, idxb1)
        lanebs = (laneb0, laneb1)
        wbs = (wb0, wb1)
        gbs = (gb0, gb1)
        encsts = (encst0, encst1)
        gsems = (gsem0, gsem1)
        esems = (esem0, esem1)

        def hash_fire(l, par):
            idxb, laneb, wb, gb = idxbs[par], lanebs[par], wbs[par], gbs[par]
            lt8 = l * (T // 4)
            scale = lax.bitcast_convert_type(
                jnp.full((LANES,), (l + 127) << 23, jnp.int32), jnp.float32)
            @plsc.parallel_loop(0, CHUNK, step=LANES)
            def _grp(p):
                xv = xyz[0, pl.ds(p, LANES)]
                yv = xyz[1, pl.ds(p, LANES)]
                zv = xyz[2, pl.ds(p, LANES)]
                px = xv * scale
                py = yv * scale
                pz = zv * scale
                ix = px.astype(jnp.uint32)
                iy = py.astype(jnp.uint32)
                iz = pz.astype(jnp.uint32)
                fx = px - ix.astype(jnp.float32)
                fy = py - iy.astype(jnp.float32)
                fz = pz - iz.astype(jnp.float32)
                one = jnp.float32(1.0)
                wxs = (one - fx, fx)
                wys = (one - fy, fy)
                wzs = (one - fz, fz)
                hx = (ix, ix + jnp.uint32(1))
                hy0 = iy * jnp.uint32(P1)
                hys = (hy0, hy0 + jnp.uint32(P1))
                hz0 = iz * jnp.uint32(P2)
                hzs = (hz0, hz0 + jnp.uint32(P2))
                for corner in range(8):
                    bx = corner & 1
                    by = (corner >> 1) & 1
                    bz = (corner >> 2) & 1
                    h = hx[bx] ^ hys[by] ^ hzs[bz]
                    hm = lax.bitcast_convert_type(
                        h & jnp.uint32(T - 1), jnp.int32)
                    w = (wxs[bx] * wys[by]) * wzs[bz]
                    row = corner * CHUNK + p
                    idxb[pl.ds(row, LANES)] = (hm >> 2) + lt8
                    laneb[pl.ds(row, LANES)] = (hm & 3) * 2
                    wb[pl.ds(row, LANES)] = w

            pltpu.async_copy(tab_hbm.at[idxb], gb, gsems[par])

        def acc_write(l, par, base):
            idxb, laneb, wb, gb = idxbs[par], lanebs[par], wbs[par], gbs[par]
            encst = encsts[par]
            enc_dst = enc_hbm.at[pl.ds(l * LEVEL_DIM, LEVEL_DIM),
                                 pl.ds(base, CHUNK)]
            @pl.when(l >= 2)
            def _():
                pltpu.make_async_copy(encst, enc_dst, esems[par]).wait()

            pltpu.make_async_copy(tab_hbm.at[idxb], gb, gsems[par]).wait()

            @plsc.parallel_loop(0, CHUNK, step=LANES)
            def _acc(p):
                acc0 = jnp.zeros((LANES,), jnp.float32)
                acc1 = jnp.zeros((LANES,), jnp.float32)
                for corner in range(8):
                    row = corner * CHUNK + p
                    rows = row + lanes
                    w = wb[pl.ds(row, LANES)]
                    lv = laneb[pl.ds(row, LANES)]
                    g0 = plsc.load_gather(gb, [rows, lv])
                    g1 = plsc.load_gather(gb, [rows, lv + 1])
                    acc0 = acc0 + w * g0
                    acc1 = acc1 + w * g1
                encst[0, pl.ds(p, LANES)] = acc0
                encst[1, pl.ds(p, LANES)] = acc1

            pltpu.async_copy(encst, enc_dst, esems[par])

        @pl.loop(0, NCHUNKS)
        def _chunk(ci):
            base = base_w + ci * CHUNK
            pltpu.sync_copy(xt_hbm.at[pl.ds(0, 3), pl.ds(base, CHUNK)], xyz)
            hash_fire(jnp.int32(0), 0)

            @pl.loop(0, NUM_LEVELS // 2)
            def _lp(lp):
                l0 = lp * 2
                hash_fire(l0 + 1, 1)
                acc_write(l0, 0, base)

                @pl.when(lp <= NUM_LEVELS // 2 - 2)
                def _():
                    hash_fire(l0 + 2, 0)

                acc_write(l0 + 1, 1, base)

            for par in range(2):
                l_last = NUM_LEVELS - 2 + par
                enc_dst = enc_hbm.at[pl.ds(l_last * LEVEL_DIM, LEVEL_DIM),
                                     pl.ds(base, CHUNK)]
                pltpu.make_async_copy(encsts[par], enc_dst,
                                      esems[par]).wait()

    return enc_kernel(xt, tabflat)


def _mlp_body(enc_ref, w0_ref, b0_ref, w1_ref, b1_ref, w2_ref, b2_ref,
              out_ref):
    e = enc_ref[...]
    dn = (((0,), (0,)), ((), ()))
    h0 = lax.dot_general(w0_ref[...], e, dn,
                         preferred_element_type=jnp.float32) + b0_ref[...]
    h0 = jnp.maximum(h0, 0.0)
    h1 = lax.dot_general(w1_ref[...], h0, dn,
                         preferred_element_type=jnp.float32) + b1_ref[...]
    h1 = jnp.maximum(h1, 0.0)
    out_ref[...] = lax.dot_general(w2_ref[...], h1, dn,
                                   preferred_element_type=jnp.float32) \
        + b2_ref[...]


def _tc_mlp(enc, W0, b0, W1, b1, W2, b2):
    n = enc.shape[1]
    grid = (n // MLP_BLOCK,)
    return pl.pallas_call(
        _mlp_body,
        grid=grid,
        in_specs=[
            pl.BlockSpec((enc.shape[0], MLP_BLOCK), lambda i: (0, i)),
            pl.BlockSpec(W0.shape, lambda i: (0, 0)),
            pl.BlockSpec((128, 1), lambda i: (0, 0)),
            pl.BlockSpec(W1.shape, lambda i: (0, 0)),
            pl.BlockSpec((128, 1), lambda i: (0, 0)),
            pl.BlockSpec(W2.shape, lambda i: (0, 0)),
            pl.BlockSpec((1, 1), lambda i: (0, 0)),
        ],
        out_specs=pl.BlockSpec((1, MLP_BLOCK), lambda i: (0, i)),
        out_shape=jax.ShapeDtypeStruct((1, n), jnp.float32),
    )(enc, W0, b0.reshape(128, 1), W1, b1.reshape(128, 1), W2,
      b2.reshape(1, 1))


@jax.jit
def kernel(x, table, W0, b0, W1, b1, W2, b2):
    xt = x.T
    tabn = (table.reshape(NUM_LEVELS, T // 128, 128, LEVEL_DIM)
            .transpose(0, 1, 3, 2).reshape(NBLK, 16))
    tabflat = _sc_relayout(tabn)
    enc = _sc_encode(xt, tabflat)
    out = _tc_mlp(enc, W0, b0, W1, b1, W2, b2)
    return out.reshape(-1, 1)
